# baseline (speedup 1.0000x reference)
import functools

import jax
import jax.numpy as jnp
from jax import lax
from jax.experimental import pallas as pl
from jax.experimental.pallas import tpu as pltpu
from jax.experimental.pallas import tpu_sc as plsc

NUM_CORES = 2
NUM_SUBCORES = 16
NW = NUM_CORES * NUM_SUBCORES
LANES = 16
IDX_CHUNK = 128


def _sc_gmf(uidx_hbm, iidx_hbm, utab_hbm, itab_hbm, w_hbm, b_hbm, out_hbm,
            uidx_v, iidx_v, u_v, i_v, w_v, b_v, out_v, sem,
            *, bpw, d):
  n_chunks = bpw // IDX_CHUNK
  wid = lax.axis_index("s") * NUM_CORES + lax.axis_index("c")
  base = wid * bpw

  pltpu.sync_copy(uidx_hbm.at[wid], uidx_v)
  pltpu.sync_copy(iidx_hbm.at[wid], iidx_v)
  pltpu.sync_copy(w_hbm, w_v)
  pltpu.sync_copy(b_hbm, b_v)

  copies = []
  for j in range(n_chunks):
    rows = pl.ds(j * IDX_CHUNK, IDX_CHUNK)
    copies.append(pltpu.async_copy(utab_hbm.at[uidx_v.at[j]], u_v.at[rows], sem))
    copies.append(pltpu.async_copy(itab_hbm.at[iidx_v.at[j]], i_v.at[rows], sem))
  for c in copies:
    c.wait()

  lanes = lax.iota(jnp.int32, LANES)
  b_vec = b_v[...]
  w_chunks = [w_v[pl.ds(k * LANES, LANES)] for k in range(d // LANES)]
  wb = [jnp.broadcast_to(w_chunks[dd // LANES][dd % LANES], (LANES,))
        for dd in range(d)]

  def body(g, carry):
    row = g * LANES + lanes
    acc = jnp.zeros((LANES,), jnp.float32)
    for dd in range(d):
      col = jnp.full((LANES,), dd, jnp.int32)
      uv = plsc.load_gather(u_v, [row, col])
      iv = plsc.load_gather(i_v, [row, col])
      acc = acc + (uv * iv) * wb[dd]
    logit = acc + b_vec
    sig = 1.0 / (1.0 + jnp.exp(-logit))
    out_v[pl.ds(g * LANES, LANES)] = sig
    return carry

  lax.fori_loop(0, bpw // LANES, body, 0)
  pltpu.sync_copy(out_v, out_hbm.at[pl.ds(base, bpw)])


def kernel(user_indices, item_indices, user_table, item_table, W, b):
  B = user_indices.shape[0]
  D = user_table.shape[1]
  bpw = B // NW
  n_chunks = bpw // IDX_CHUNK

  uidx = user_indices.astype(jnp.int32).reshape(NW, n_chunks, IDX_CHUNK)
  iidx = item_indices.astype(jnp.int32).reshape(NW, n_chunks, IDX_CHUNK)
  w_flat = W.reshape(D).astype(jnp.float32)
  b_vec = jnp.broadcast_to(b.astype(jnp.float32), (LANES,))

  mesh = plsc.VectorSubcoreMesh(core_axis_name="c", subcore_axis_name="s")
  sc = functools.partial(
      pl.kernel,
      mesh=mesh,
      compiler_params=pltpu.CompilerParams(
          needs_layout_passes=False, use_tc_tiling_on_sc=False),
      out_type=jax.ShapeDtypeStruct((B,), jnp.float32),
      scratch_types=[
          pltpu.VMEM((n_chunks, IDX_CHUNK), jnp.int32),
          pltpu.VMEM((n_chunks, IDX_CHUNK), jnp.int32),
          pltpu.VMEM((bpw, D), jnp.float32),
          pltpu.VMEM((bpw, D), jnp.float32),
          pltpu.VMEM((D,), jnp.float32),
          pltpu.VMEM((LANES,), jnp.float32),
          pltpu.VMEM((bpw,), jnp.float32),
          pltpu.SemaphoreType.DMA,
      ],
  )(functools.partial(_sc_gmf, bpw=bpw, d=D))

  out = sc(uidx, iidx, user_table, item_table, w_flat, b_vec)
  return out.reshape(B, 1)
